# baseline (speedup 1.0000x reference)
import functools

import jax
import jax.numpy as jnp
from jax import lax
from jax.experimental import pallas as pl
from jax.experimental.pallas import tpu as pltpu
from jax.experimental.pallas import tpu_sc as plsc

NUM_EMB = 1024
EMB_DIM = 64
ROW_BLK = 1024


IMGS_PER_STEP = 8


def _argmin_q_kernel(x_ref, emb_ref, out_ref):
    xc = x_ref[...]
    emb = emb_ref[...]
    imgs, _, hw = xc.shape
    x = jnp.transpose(xc, (0, 2, 1)).reshape(imgs * hw, EMB_DIM)
    a = jnp.sum(x * x, axis=1, keepdims=True)
    b = jnp.sum(emb * emb, axis=1)
    c = lax.dot_general(x, emb, (((1,), (1,)), ((), ())),
                        preferred_element_type=jnp.float32)
    dist = (a + b[None, :]) - 2.0 * c
    m = jnp.min(dist, axis=1, keepdims=True)
    ii = lax.broadcasted_iota(jnp.int32, (1, NUM_EMB), 1).astype(jnp.float32)
    idx = jnp.min(jnp.where(dist == m, ii, float(NUM_EMB)), axis=1)
    iie = lax.broadcasted_iota(jnp.int32, (NUM_EMB, 1), 0)
    onehot_t = (iie == idx.astype(jnp.int32)[None, :]).astype(jnp.bfloat16)
    q = lax.dot_general(emb.astype(jnp.bfloat16), onehot_t,
                        (((0,), (0,)), ((), ())),
                        preferred_element_type=jnp.float32)
    for im in range(imgs):
        out_ref[im] = q[:, im * hw:(im + 1) * hw]


def _quantize_nchw(x_nc_hw, embedding):
    n, ch, hw = x_nc_hw.shape
    g = n // IMGS_PER_STEP
    return pl.pallas_call(
        _argmin_q_kernel,
        grid=(g,),
        in_specs=[
            pl.BlockSpec((IMGS_PER_STEP, EMB_DIM, hw), lambda i: (i, 0, 0)),
            pl.BlockSpec((NUM_EMB, EMB_DIM), lambda i: (0, 0)),
        ],
        out_specs=pl.BlockSpec((IMGS_PER_STEP, ch, hw), lambda i: (i, 0, 0)),
        out_shape=jax.ShapeDtypeStruct((n, ch, hw), jnp.float32),
    )(x_nc_hw, embedding)


@functools.lru_cache(maxsize=None)
def _make_sc_gather(v, d, b):
    info = plsc.get_sparse_core_info()
    nc, ns = info.num_cores, info.num_subcores
    nw = nc * ns
    assert d % info.num_lanes == 0 and b % (8 * nw) == 0
    b_per_w = b // nw
    mesh = plsc.VectorSubcoreMesh(core_axis_name="c", subcore_axis_name="s")

    @functools.partial(
        pl.kernel, mesh=mesh,
        compiler_params=pltpu.CompilerParams(use_tc_tiling_on_sc=False),
        out_type=jax.ShapeDtypeStruct((b, d), jnp.float32),
        scratch_types=[
            pltpu.VMEM((b_per_w,), jnp.int32),
            pltpu.VMEM((b_per_w, d), jnp.float32),
            pltpu.SemaphoreType.DMA,
        ],
    )
    def gather(table_hbm, idx_hbm, out_hbm, idx_v, rows_v, sem):
        wid = lax.axis_index("s") * nc + lax.axis_index("c")
        base = wid * b_per_w
        pltpu.sync_copy(idx_hbm.at[pl.ds(base, b_per_w)], idx_v)
        pltpu.async_copy(table_hbm.at[idx_v], rows_v, sem).wait()
        pltpu.sync_copy(rows_v, out_hbm.at[pl.ds(base, b_per_w)])

    return gather


def _transpose_kernel(rows_ref, out_ref):
    out_ref[0] = rows_ref[0].T


def _rows_to_nchw(rows, n, ch, hw):
    return pl.pallas_call(
        _transpose_kernel,
        grid=(n,),
        in_specs=[pl.BlockSpec((1, hw, ch), lambda i: (i, 0, 0))],
        out_specs=pl.BlockSpec((1, ch, hw), lambda i: (i, 0, 0)),
        out_shape=jax.ShapeDtypeStruct((n, ch, hw), jnp.float32),
    )(rows.reshape(n, hw, ch))


def kernel(inputs, embedding):
    n, ch, h, w = inputs.shape
    x_nc_hw = inputs.reshape(n, ch, h * w)
    out = _quantize_nchw(x_nc_hw, embedding)
    return out.reshape(n, ch, h, w)


# speedup vs baseline: 1.0087x; 1.0087x over previous
import jax
import jax.numpy as jnp
from jax import lax
from jax.experimental import pallas as pl

NUM_EMB = 1024
EMB_DIM = 64
IMGS_PER_STEP = 4


def _argmin_q_kernel(x_ref, emb_ref, out_ref):
    xc = x_ref[...]
    emb = emb_ref[...]
    imgs, _, hw = xc.shape
    x = jnp.transpose(xc, (0, 2, 1)).reshape(imgs * hw, EMB_DIM)
    a = jnp.sum(x * x, axis=1, keepdims=True)
    b = jnp.sum(emb * emb, axis=1)
    c = lax.dot_general(x, emb, (((1,), (1,)), ((), ())),
                        preferred_element_type=jnp.float32)
    dist = (a + b[None, :]) - 2.0 * c
    m = jnp.min(dist, axis=1, keepdims=True)
    ii = lax.broadcasted_iota(jnp.int32, (1, NUM_EMB), 1).astype(jnp.float32)
    idx = jnp.min(jnp.where(dist == m, ii, float(NUM_EMB)), axis=1)
    iie = lax.broadcasted_iota(jnp.int32, (NUM_EMB, 1), 0)
    onehot_t = (iie == idx.astype(jnp.int32)[None, :]).astype(jnp.bfloat16)
    q = lax.dot_general(emb.astype(jnp.bfloat16), onehot_t,
                        (((0,), (0,)), ((), ())),
                        preferred_element_type=jnp.float32)
    for im in range(imgs):
        out_ref[im] = q[:, im * hw:(im + 1) * hw]


def _quantize_nchw(x_nc_hw, embedding):
    n, ch, hw = x_nc_hw.shape
    g = n // IMGS_PER_STEP
    return pl.pallas_call(
        _argmin_q_kernel,
        grid=(g,),
        in_specs=[
            pl.BlockSpec((IMGS_PER_STEP, EMB_DIM, hw), lambda i: (i, 0, 0)),
            pl.BlockSpec((NUM_EMB, EMB_DIM), lambda i: (0, 0)),
        ],
        out_specs=pl.BlockSpec((IMGS_PER_STEP, ch, hw), lambda i: (i, 0, 0)),
        out_shape=jax.ShapeDtypeStruct((n, ch, hw), jnp.float32),
    )(x_nc_hw, embedding)


def kernel(inputs, embedding):
    n, ch, h, w = inputs.shape
    x_nc_hw = inputs.reshape(n, ch, h * w)
    out = _quantize_nchw(x_nc_hw, embedding)
    return out.reshape(n, ch, h, w)
